# Initial kernel scaffold; baseline (speedup 1.0000x reference)
import jax
import jax.numpy as jnp
from jax.experimental import pallas as pl


def kernel(x, x_initial, edge_index, adj_values, w_init, w_x):
    raise NotImplementedError("write your pallas kernel here")


# speedup vs baseline: 4.7474x; 4.7474x over previous
import functools

import jax
import jax.numpy as jnp
from jax import lax
from jax.experimental import pallas as pl
from jax.experimental.pallas import tpu as pltpu
from jax.experimental.pallas import tpu_sc as plsc

_N = 10000
_E = 320000
_D = 128
_ALPHA = 0.9

_NC = 2
_NS = 16
_NW = _NC * _NS
_CH = 128
_NCH = 79
_PER_W = _NCH * _CH
_E_PAD = _NW * _PER_W
_N_PAD = 10240
_RPW = _N_PAD // _NS
_ZB = 128


def _sc_body(x_hbm, src_hbm, dst_hbm, val_hbm, out_hbm,
             agg_sh, src_v, dst_v, val_v, rows_v):
    c = lax.axis_index("c")
    s = lax.axis_index("s")
    w = c * _NS + s

    def zset(i, carry):
        z = jnp.zeros((16,), jnp.float32)
        for j in range(_D // 16):
            rows_v[i, pl.ds(j * 16, 16)] = z
        return carry

    lax.fori_loop(0, _ZB, zset, 0)

    def zdma(k, carry):
        pltpu.sync_copy(rows_v, agg_sh.at[pl.ds(s * _RPW + k * _ZB, _ZB)])
        return carry

    lax.fori_loop(0, _RPW // _ZB, zdma, 0)

    pltpu.sync_copy(src_hbm.at[w], src_v)
    pltpu.sync_copy(dst_hbm.at[w], dst_v)
    pltpu.sync_copy(val_hbm.at[w], val_v)

    plsc.subcore_barrier()

    def chunk(i, carry):
        pltpu.sync_copy(x_hbm.at[src_v.at[i]], rows_v)

        def group(g, inner):
            vv16 = val_v[i, pl.ds(g * 16, 16)]
            for e16 in range(16):
                bc = jnp.take_along_axis(
                    vv16, jnp.full((16,), e16, jnp.int32), axis=0)
                e = g * 16 + e16
                for j in range(_D // 16):
                    sl = pl.ds(j * 16, 16)
                    rows_v[e, sl] = rows_v[e, sl] * bc
            return inner

        lax.fori_loop(0, _CH // 16, group, 0)
        pltpu.sync_copy(rows_v, agg_sh.at[dst_v.at[i]], add=True)
        return carry

    lax.fori_loop(0, _NCH, chunk, 0)

    plsc.subcore_barrier()

    pltpu.sync_copy(agg_sh.at[pl.ds(s * _RPW, _RPW)],
                    out_hbm.at[c, pl.ds(s * _RPW, _RPW)])


_sc_gather_scatter = functools.partial(
    pl.kernel,
    out_type=jax.ShapeDtypeStruct((_NC, _N_PAD, _D), jnp.float32),
    mesh=plsc.VectorSubcoreMesh(core_axis_name="c", subcore_axis_name="s"),
    scratch_types=[
        pltpu.VMEM_SHARED((_N_PAD, _D), jnp.float32),
        pltpu.VMEM((_NCH, _CH), jnp.int32),
        pltpu.VMEM((_NCH, _CH), jnp.int32),
        pltpu.VMEM((_NCH, _CH), jnp.float32),
        pltpu.VMEM((_CH, _D), jnp.float32),
    ],
)(_sc_body)


_BLK = 400


def _tc_body(p_ref, xi_ref, wi_ref, wx_ref, o_ref):
    agg = p_ref[0] + p_ref[1]
    h = _ALPHA * agg + (1.0 - _ALPHA) * jnp.dot(
        xi_ref[...], wi_ref[...], preferred_element_type=jnp.float32)
    o_ref[...] = jnp.maximum(
        jnp.dot(h, wx_ref[...], preferred_element_type=jnp.float32), 0.0)


def _tc_dense(partials, xi, wi, wx):
    nblk = _N // _BLK
    return pl.pallas_call(
        _tc_body,
        out_shape=jax.ShapeDtypeStruct((_N, _D), jnp.float32),
        grid=(nblk,),
        in_specs=[
            pl.BlockSpec((2, _BLK, _D), lambda i: (0, i, 0)),
            pl.BlockSpec((_BLK, 8), lambda i: (i, 0)),
            pl.BlockSpec((8, _D), lambda i: (0, 0)),
            pl.BlockSpec((_D, _D), lambda i: (0, 0)),
        ],
        out_specs=pl.BlockSpec((_BLK, _D), lambda i: (i, 0)),
    )(partials, xi, wi, wx)


def kernel(x, x_initial, edge_index, adj_values, w_init, w_x):
    dst = edge_index[0]
    src = edge_index[1]
    pad = _E_PAD - _E
    zi = jnp.zeros((pad,), jnp.int32)
    srcp = jnp.concatenate([src, zi]).reshape(_NW, _NCH, _CH)
    dstp = jnp.concatenate([dst, zi]).reshape(_NW, _NCH, _CH)
    valp = jnp.concatenate(
        [adj_values, jnp.zeros((pad,), jnp.float32)]).reshape(_NW, _NCH, _CH)

    partials = _sc_gather_scatter(x, srcp, dstp, valp)

    xi = jnp.pad(x_initial, ((0, 0), (0, 5)))
    wi = jnp.pad(w_init, ((0, 5), (0, 0)))
    return _tc_dense(partials, xi, wi, w_x)
